# baseline (speedup 1.0000x reference)
import functools

import jax
import jax.numpy as jnp
from jax import lax
from jax.experimental import pallas as pl
from jax.experimental.pallas import tpu as pltpu
from jax.experimental.pallas import tpu_sc as plsc

N = 10000
E = 320000
F = 128
K = 80
EPT = E // 16
ACCN = 10240
RPT = ACCN // 16
DEN = ACCN // F


_GD = lax.GatherDimensionNumbers(
    offset_dims=(), collapsed_slice_dims=(0,), start_index_map=(0,)
)


def _permute(x, idx16):
    return lax.gather(
        x, idx16[:, None], _GD, (1,),
        mode=lax.GatherScatterMode.PROMISE_IN_BOUNDS,
    )


def _pre_body(h_ref, w_ref, hhs_ref):
    h = h_ref[...]
    w = w_ref[...]
    nrm = jnp.sqrt(jnp.sum(w * w))
    wn = w / jnp.maximum(nrm, 1e-12)
    p = lax.dot_general(h, wn, (((1,), (1,)), ((), ())))
    hhs_ref[0:N, :] = h
    hhs_ref[N : 2 * N, :] = h - 2.0 * p * wn


_pre_call = pl.pallas_call(
    _pre_body,
    out_shape=jax.ShapeDtypeStruct((2 * N, F), jnp.float32),
)


@functools.partial(
    pl.kernel,
    mesh=plsc.VectorSubcoreMesh(core_axis_name="c", subcore_axis_name="s"),
    out_type=jax.ShapeDtypeStruct((2 * N, F), jnp.float32),
    scratch_types=[
        pltpu.VMEM((3 * K,), jnp.int32),
        pltpu.VMEM((3 * K,), jnp.int32),
        pltpu.VMEM((K,), jnp.int32),
        pltpu.VMEM((K,), jnp.int32),
        pltpu.VMEM((K,), jnp.int32),
        pltpu.VMEM((K,), jnp.int32),
        pltpu.VMEM((K,), jnp.int32),
        pltpu.VMEM((K,), jnp.int32),
        pltpu.VMEM((K,), jnp.int32),
        pltpu.VMEM((K,), jnp.int32),
        pltpu.VMEM((F,), jnp.float32),
        pltpu.VMEM((F,), jnp.float32),
        pltpu.VMEM((K, F), jnp.float32),
        pltpu.VMEM((K, F), jnp.float32),
        pltpu.VMEM((K, F), jnp.float32),
        pltpu.VMEM((K * 16,), jnp.float32),
        pltpu.VMEM_SHARED((ACCN, F), jnp.float32),
        pltpu.VMEM_SHARED((DEN, F), jnp.float32),
        pltpu.SemaphoreType.DMA,
        pltpu.SemaphoreType.DMA,
        pltpu.SemaphoreType.DMA,
        pltpu.SemaphoreType.DMA,
    ],
)
def _edge_kernel(
    hhs_hbm, r_hbm, amat_hbm, pk_hbm, out_hbm,
    pkA, pkB, dstA, dstB, relA, relB, srcA, srcB, dsgA, dsgB,
    a0_v, a1_v, hrA, hrB, rr,
    w_buf, acc, den_sh, semHA, semHB, semR, semX,
):
    c_id = lax.axis_index("c")
    s_id = lax.axis_index("s")
    coff = c_id * N
    lane = lax.iota(jnp.int32, 16)
    zero16 = jnp.zeros((16,), jnp.float32)
    pltpu.sync_copy(amat_hbm.at[pl.ds(c_id * F, F)], a0_v)
    pltpu.sync_copy(amat_hbm.at[pl.ds((c_id + 2) * F, F)], a1_v)
    a0s = [a0_v[pl.ds(f * 16, 16)] for f in range(F // 16)]
    a1s = [a1_v[pl.ds(f * 16, 16)] for f in range(F // 16)]
    def zv(k, _):
        for f in range(F // 16):
            hrA[k, pl.ds(f * 16, 16)] = zero16
        return 0

    lax.fori_loop(0, K, zv, 0)
    row0 = s_id * RPT
    for t in range(RPT // K):
        pltpu.sync_copy(hrA, acc.at[pl.ds(row0 + t * K, K)])
    pl.when(s_id == 0)(lambda: pltpu.sync_copy(hrA, den_sh))
    plsc.subcore_barrier()

    NCH = EPT // K

    def prefetch(j, pk, dst, dsg, rel, srcb, hr, semH):
        p0 = (s_id * NCH + j) * (3 * K)
        pltpu.sync_copy(pk_hbm.at[pl.ds(p0, 3 * K)], pk)

        def gi(i, _):
            sl = pl.ds(i * 16, 16)
            d16i = pk[sl]
            dst[sl] = d16i
            rel[sl] = pk[pl.ds(K + i * 16, 16)]
            srcb[sl] = pk[pl.ds(2 * K + i * 16, 16)] + coff
            dsg[sl] = lax.shift_right_logical(d16i, 7)
            return 0

        lax.fori_loop(0, K // 16, gi, 0)
        pltpu.async_copy(hhs_hbm.at[srcb], hr, semH)

    def wait_h(hr, semH):
        pltpu.make_async_copy(hhs_hbm.at[pl.ds(0, K)], hr, semH).wait()

    def wait_r():
        pltpu.make_async_copy(r_hbm.at[pl.ds(0, K)], rr, semR).wait()

    def compute(dst, dsg, hrows, rrows):
        def wi(k, _):
            hs = [hrows[k, pl.ds(f * 16, 16)] for f in range(F // 16)]
            rs = [rrows[k, pl.ds(f * 16, 16)] for f in range(F // 16)]
            d16 = hs[0] * a0s[0] + rs[0] * a1s[0]
            for f in range(1, F // 16):
                d16 = d16 + hs[f] * a0s[f] + rs[f] * a1s[f]
            x = d16
            for sft in (8, 4, 2, 1):
                perm = jnp.bitwise_and(lane + sft, 15)
                x = x + _permute(x, perm)
            lr = jnp.where(x > 0.0, x, 0.2 * x)
            wv = jnp.exp(-lr)
            w_buf[pl.ds(k * 16, 16)] = wv
            for f in range(F // 16):
                hrows[k, pl.ds(f * 16, 16)] = (hs[f] - rs[f]) * wv
            return 0

        lax.fori_loop(0, K, wi, 0, unroll=2)
        pltpu.sync_copy(hrows, acc.at[dst], add=True)

        def oi(g, _):
            dst16 = dst[pl.ds(g * 16, 16)]
            col16 = jnp.bitwise_and(dst16, 127)
            for jj in range(16):
                k = g * 16 + jj
                colr = _permute(col16, jnp.full((16,), jj, jnp.int32))
                wr = w_buf[pl.ds(k * 16, 16)]
                for f in range(F // 16):
                    hrows[k, pl.ds(f * 16, 16)] = jnp.where(
                        colr == lane + f * 16, wr, 0.0
                    )
            return 0

        lax.fori_loop(0, K // 16, oi, 0)
        pltpu.sync_copy(hrows, den_sh.at[dsg], add=True)

    prefetch(0, pkA, dstA, dsgA, relA, srcA, hrA, semHA)

    def pipe(i, _):
        wait_h(hrA, semHA)
        pltpu.async_copy(r_hbm.at[relA], rr, semR)
        prefetch(2 * i + 1, pkB, dstB, dsgB, relB, srcB, hrB, semHB)
        wait_r()
        compute(dstA, dsgA, hrA, rr)
        wait_h(hrB, semHB)
        pltpu.async_copy(r_hbm.at[relB], rr, semR)
        pl.when(i < NCH // 2 - 1)(
            lambda: prefetch(2 * i + 2, pkA, dstA, dsgA, relA, srcA,
                             hrA, semHA)
        )
        wait_r()
        compute(dstB, dsgB, hrB, rr)
        return 0

    lax.fori_loop(0, NCH // 2, pipe, 0)
    plsc.subcore_barrier()

    denb = jnp.minimum(jnp.bitwise_and(s_id * 5, -8), DEN - 16)
    pltpu.sync_copy(den_sh.at[pl.ds(denb, 16)], hrB.at[pl.ds(0, 16)])

    def norm_chunk(r0_):
        pltpu.sync_copy(acc.at[pl.ds(r0_, K)], rr)

        def ni(g, _):
            n0 = r0_ + g * 16
            den16 = hrB[
                lax.shift_right_logical(n0, 7) - denb,
                pl.ds(jnp.bitwise_and(n0, 127), 16),
            ]
            for jj in range(16):
                k = g * 16 + jj
                denr = _permute(den16, jnp.full((16,), jj, jnp.int32))
                for f in range(F // 16):
                    sl = pl.ds(f * 16, 16)
                    hrA[k, sl] = rr[k, sl] / denr
            return 0

        lax.fori_loop(0, K // 16, ni, 0)
        pltpu.sync_copy(hrA, out_hbm.at[pl.ds(coff + r0_, K)])

    for t in range(RPT // K):
        r0_ = row0 + t * K
        pl.when(r0_ < N)(lambda: norm_chunk(r0_))


def kernel(h, inputr, A, w_ori, a_src_dst):
    avec = a_src_dst[:, :, :, 0]
    amat = jnp.stack(
        [avec[0, 0], avec[1, 0], avec[0, 1], avec[1, 1]], axis=0
    ).reshape(-1)
    hhs = _pre_call(h, w_ori)
    packed = jnp.transpose(
        A.reshape(3, 16, EPT // K, K), (1, 2, 0, 3)
    ).reshape(-1)
    out_flat = _edge_kernel(hhs, inputr, amat, packed)
    return out_flat.reshape(2, N, F)


# speedup vs baseline: 1.0429x; 1.0429x over previous
import functools

import jax
import jax.numpy as jnp
from jax import lax
from jax.experimental import pallas as pl
from jax.experimental.pallas import tpu as pltpu
from jax.experimental.pallas import tpu_sc as plsc

N = 10000
E = 320000
F = 128
K = 80
EPT = E // 16
ACCN = 10240
RPT = ACCN // 16
DEN = ACCN // F


_GD = lax.GatherDimensionNumbers(
    offset_dims=(), collapsed_slice_dims=(0,), start_index_map=(0,)
)


def _permute(x, idx16):
    return lax.gather(
        x, idx16[:, None], _GD, (1,),
        mode=lax.GatherScatterMode.PROMISE_IN_BOUNDS,
    )


def _pre_body(h_ref, w_ref, hhs_ref):
    h = h_ref[...]
    w = w_ref[...]
    nrm = jnp.sqrt(jnp.sum(w * w))
    wn = w / jnp.maximum(nrm, 1e-12)
    p = lax.dot_general(h, wn, (((1,), (1,)), ((), ())))
    hhs_ref[0:N, :] = h
    hhs_ref[N : 2 * N, :] = h - 2.0 * p * wn


_pre_call = pl.pallas_call(
    _pre_body,
    out_shape=jax.ShapeDtypeStruct((2 * N, F), jnp.float32),
)


@functools.partial(
    pl.kernel,
    mesh=plsc.VectorSubcoreMesh(core_axis_name="c", subcore_axis_name="s"),
    out_type=jax.ShapeDtypeStruct((2 * N, F), jnp.float32),
    scratch_types=[
        pltpu.VMEM((3 * K,), jnp.int32),
        pltpu.VMEM((3 * K,), jnp.int32),
        pltpu.VMEM((K,), jnp.int32),
        pltpu.VMEM((K,), jnp.int32),
        pltpu.VMEM((K,), jnp.int32),
        pltpu.VMEM((K,), jnp.int32),
        pltpu.VMEM((K,), jnp.int32),
        pltpu.VMEM((K,), jnp.int32),
        pltpu.VMEM((K,), jnp.int32),
        pltpu.VMEM((K,), jnp.int32),
        pltpu.VMEM((F,), jnp.float32),
        pltpu.VMEM((F,), jnp.float32),
        pltpu.VMEM((K, F), jnp.float32),
        pltpu.VMEM((K, F), jnp.float32),
        pltpu.VMEM((K, F), jnp.float32),
        pltpu.VMEM((K * 16,), jnp.float32),
        pltpu.VMEM_SHARED((ACCN, F), jnp.float32),
        pltpu.VMEM_SHARED((DEN, F), jnp.float32),
        pltpu.SemaphoreType.DMA,
        pltpu.SemaphoreType.DMA,
        pltpu.SemaphoreType.DMA,
        pltpu.SemaphoreType.DMA,
    ],
)
def _edge_kernel(
    hhs_hbm, r_hbm, amat_hbm, pk_hbm, out_hbm,
    pkA, pkB, dstA, dstB, relA, relB, srcA, srcB, dsgA, dsgB,
    a0_v, a1_v, hrA, hrB, rr,
    w_buf, acc, den_sh, semHA, semHB, semR, semX,
):
    c_id = lax.axis_index("c")
    s_id = lax.axis_index("s")
    coff = c_id * N
    lane = lax.iota(jnp.int32, 16)
    zero16 = jnp.zeros((16,), jnp.float32)
    pltpu.sync_copy(amat_hbm.at[pl.ds(c_id * F, F)], a0_v)
    pltpu.sync_copy(amat_hbm.at[pl.ds((c_id + 2) * F, F)], a1_v)
    a0s = [a0_v[pl.ds(f * 16, 16)] for f in range(F // 16)]
    a1s = [a1_v[pl.ds(f * 16, 16)] for f in range(F // 16)]
    def zv(k, _):
        for f in range(F // 16):
            hrA[k, pl.ds(f * 16, 16)] = zero16
        return 0

    lax.fori_loop(0, K, zv, 0)
    row0 = s_id * RPT
    for t in range(RPT // K):
        pltpu.sync_copy(hrA, acc.at[pl.ds(row0 + t * K, K)])
    pl.when(s_id == 0)(lambda: pltpu.sync_copy(hrA, den_sh))
    plsc.subcore_barrier()

    NCH = EPT // K

    def prefetch(j, pk, dst, dsg, rel, srcb, hr, semH):
        p0 = (s_id * NCH + j) * (3 * K)
        pltpu.sync_copy(pk_hbm.at[pl.ds(p0, 3 * K)], pk)

        def gi(i, _):
            sl = pl.ds(i * 16, 16)
            d16i = pk[sl]
            dst[sl] = d16i
            rel[sl] = pk[pl.ds(K + i * 16, 16)]
            srcb[sl] = pk[pl.ds(2 * K + i * 16, 16)] + coff
            dsg[sl] = lax.shift_right_logical(d16i, 7)
            return 0

        lax.fori_loop(0, K // 16, gi, 0)
        pltpu.async_copy(hhs_hbm.at[srcb], hr, semH)

    def wait_h(hr, semH):
        pltpu.make_async_copy(hhs_hbm.at[pl.ds(0, K)], hr, semH).wait()

    def wait_r():
        pltpu.make_async_copy(r_hbm.at[pl.ds(0, K)], rr, semR).wait()

    def compute(dst, dsg, hrows, rrows):
        def wi(k, _):
            hs = [hrows[k, pl.ds(f * 16, 16)] for f in range(F // 16)]
            rs = [rrows[k, pl.ds(f * 16, 16)] for f in range(F // 16)]
            d16 = hs[0] * a0s[0] + rs[0] * a1s[0]
            for f in range(1, F // 16):
                d16 = d16 + hs[f] * a0s[f] + rs[f] * a1s[f]
            x = d16
            for sft in (8, 4, 2, 1):
                perm = jnp.bitwise_and(lane + sft, 15)
                x = x + _permute(x, perm)
            lr = jnp.where(x > 0.0, x, 0.2 * x)
            wv = jnp.exp(-lr)
            w_buf[pl.ds(k * 16, 16)] = wv
            for f in range(F // 16):
                hrows[k, pl.ds(f * 16, 16)] = (hs[f] - rs[f]) * wv
            return 0

        lax.fori_loop(0, K, wi, 0)
        pltpu.sync_copy(hrows, acc.at[dst], add=True)

        def oi(g, _):
            dst16 = dst[pl.ds(g * 16, 16)]
            col16 = jnp.bitwise_and(dst16, 127)
            for jj in range(16):
                k = g * 16 + jj
                colr = _permute(col16, jnp.full((16,), jj, jnp.int32))
                wr = w_buf[pl.ds(k * 16, 16)]
                for f in range(F // 16):
                    hrows[k, pl.ds(f * 16, 16)] = jnp.where(
                        colr == lane + f * 16, wr, 0.0
                    )
            return 0

        lax.fori_loop(0, K // 16, oi, 0)
        pltpu.sync_copy(hrows, den_sh.at[dsg], add=True)

    prefetch(0, pkA, dstA, dsgA, relA, srcA, hrA, semHA)

    def pipe(i, _):
        wait_h(hrA, semHA)
        pltpu.async_copy(r_hbm.at[relA], rr, semR)
        prefetch(2 * i + 1, pkB, dstB, dsgB, relB, srcB, hrB, semHB)
        wait_r()
        compute(dstA, dsgA, hrA, rr)
        wait_h(hrB, semHB)
        pltpu.async_copy(r_hbm.at[relB], rr, semR)
        pl.when(i < NCH // 2 - 1)(
            lambda: prefetch(2 * i + 2, pkA, dstA, dsgA, relA, srcA,
                             hrA, semHA)
        )
        wait_r()
        compute(dstB, dsgB, hrB, rr)
        return 0

    lax.fori_loop(0, NCH // 2, pipe, 0)
    plsc.subcore_barrier()

    denb = jnp.minimum(jnp.bitwise_and(s_id * 5, -8), DEN - 16)
    pltpu.sync_copy(den_sh.at[pl.ds(denb, 16)], hrB.at[pl.ds(0, 16)])

    def norm_chunk(r0_):
        pltpu.sync_copy(acc.at[pl.ds(r0_, K)], rr)

        def ni(g, _):
            n0 = r0_ + g * 16
            den16 = hrB[
                lax.shift_right_logical(n0, 7) - denb,
                pl.ds(jnp.bitwise_and(n0, 127), 16),
            ]
            for jj in range(16):
                k = g * 16 + jj
                denr = _permute(den16, jnp.full((16,), jj, jnp.int32))
                for f in range(F // 16):
                    sl = pl.ds(f * 16, 16)
                    hrA[k, sl] = rr[k, sl] / denr
            return 0

        lax.fori_loop(0, K // 16, ni, 0)
        pltpu.sync_copy(hrA, out_hbm.at[pl.ds(coff + r0_, K)])

    for t in range(RPT // K):
        r0_ = row0 + t * K
        pl.when(r0_ < N)(lambda: norm_chunk(r0_))


def kernel(h, inputr, A, w_ori, a_src_dst):
    avec = a_src_dst[:, :, :, 0]
    amat = jnp.stack(
        [avec[0, 0], avec[1, 0], avec[0, 1], avec[1, 1]], axis=0
    ).reshape(-1)
    hhs = _pre_call(h, w_ori)
    packed = jnp.transpose(
        A.reshape(3, 16, EPT // K, K), (1, 2, 0, 3)
    ).reshape(-1)
    out_flat = _edge_kernel(hhs, inputr, amat, packed)
    return out_flat.reshape(2, N, F)
